# baseline (speedup 1.0000x reference)
import functools

import jax
import jax.numpy as jnp
from jax import lax
from jax.experimental import pallas as pl
from jax.experimental.pallas import tpu as pltpu
from jax.experimental.pallas import tpu_sc as plsc

NUM_FACTORS = 128
BATCH = 16384

_SC_INFO = plsc.get_sparse_core_info()
_NC = _SC_INFO.num_cores
_NS = _SC_INFO.num_subcores
_NW = _NC * _NS
_B_PER_W = BATCH // _NW
_CHUNK = 64
_NCHUNKS = _B_PER_W // _CHUNK
_LANES = 16
_VPR = NUM_FACTORS // _LANES


def _sc_gather_body(uid, iid, p_hbm, q_hbm, u_hbm, v_hbm,
                    og, ou, ov,
                    idx_u0, idx_i0, bp0, bq0, bu0, bv0,
                    idx_u1, idx_i1, bp1, bq1, bu1, bv1,
                    gsem0, gsem1, ssem0, ssem1):
    idx_u = (idx_u0, idx_u1)
    idx_i = (idx_i0, idx_i1)
    bp = (bp0, bp1)
    bq = (bq0, bq1)
    bu = (bu0, bu1)
    bv = (bv0, bv1)
    gsem = (gsem0, gsem1)
    ssem = (ssem0, ssem1)

    wid = lax.axis_index("s") * _NC + lax.axis_index("c")
    base = wid * _B_PER_W

    def issue_gathers(c, d):
        row0 = base + c * _CHUNK
        pltpu.sync_copy(uid.at[pl.ds(row0, _CHUNK)], idx_u[d])
        pltpu.sync_copy(iid.at[pl.ds(row0, _CHUNK)], idx_i[d])
        return (
            pltpu.async_copy(p_hbm.at[idx_u[d]], bp[d], gsem[d]),
            pltpu.async_copy(q_hbm.at[idx_i[d]], bq[d], gsem[d]),
            pltpu.async_copy(u_hbm.at[idx_u[d]], bu[d], gsem[d]),
            pltpu.async_copy(v_hbm.at[idx_i[d]], bv[d], gsem[d]),
        )

    def issue_scatters(c, d):
        row0 = base + c * _CHUNK
        return (
            pltpu.async_copy(bp[d], og.at[pl.ds(row0, _CHUNK)], ssem[d]),
            pltpu.async_copy(bu[d], ou.at[pl.ds(row0, _CHUNK)], ssem[d]),
            pltpu.async_copy(bv[d], ov.at[pl.ds(row0, _CHUNK)], ssem[d]),
        )

    gcopies = [None, None]
    scopies = [None, None]
    gcopies[0] = issue_gathers(0, 0)
    gcopies[1] = issue_gathers(1, 1)
    for c in range(_NCHUNKS):
        d = c % 2
        for cp in gcopies[d]:
            cp.wait()

        def mul_row(r, _, d=d):
            for j in range(_VPR):
                s = pl.ds(j * _LANES, _LANES)
                bp[d][r, s] = bp[d][r, s] * bq[d][r, s]
            return 0

        lax.fori_loop(0, _CHUNK, mul_row, 0, unroll=2)
        scopies[d] = issue_scatters(c, d)
        if c + 2 < _NCHUNKS:
            for cp in scopies[d]:
                cp.wait()
            gcopies[d] = issue_gathers(c + 2, d)
    for d in range(2):
        for cp in scopies[d]:
            cp.wait()


_ROW_SHAPE = jax.ShapeDtypeStruct((BATCH, NUM_FACTORS), jnp.float32)


def _row_bufs():
    return [
        pltpu.VMEM((_CHUNK,), jnp.int32),
        pltpu.VMEM((_CHUNK,), jnp.int32),
        pltpu.VMEM((_CHUNK, NUM_FACTORS), jnp.float32),
        pltpu.VMEM((_CHUNK, NUM_FACTORS), jnp.float32),
        pltpu.VMEM((_CHUNK, NUM_FACTORS), jnp.float32),
        pltpu.VMEM((_CHUNK, NUM_FACTORS), jnp.float32),
    ]


_sc_gather = functools.partial(
    pl.kernel,
    mesh=plsc.VectorSubcoreMesh(core_axis_name="c", subcore_axis_name="s"),
    out_type=(_ROW_SHAPE, _ROW_SHAPE, _ROW_SHAPE),
    scratch_types=_row_bufs() + _row_bufs() + [
        pltpu.SemaphoreType.DMA,
        pltpu.SemaphoreType.DMA,
        pltpu.SemaphoreType.DMA,
        pltpu.SemaphoreType.DMA,
    ],
)(_sc_gather_body)


_R = 2048


def _tc_mlp_body(gm, um, vm, w1, b1, w2, b2, w3, b3, wo, out):
    bf = jnp.bfloat16
    xu = um[...].astype(bf)
    xv = vm[...].astype(bf)
    w1v = w1[...].astype(bf)
    h1 = jnp.dot(xu, w1v[:NUM_FACTORS], preferred_element_type=jnp.float32)
    h1 += jnp.dot(xv, w1v[NUM_FACTORS:], preferred_element_type=jnp.float32)
    h1 = jnp.maximum(h1 + b1[...][None, :], 0.0).astype(bf)
    h2 = jnp.dot(h1, w2[...].astype(bf), preferred_element_type=jnp.float32)
    h2 = jnp.maximum(h2 + b2[...][None, :], 0.0).astype(bf)
    h3 = jnp.dot(h2, w3[...].astype(bf), preferred_element_type=jnp.float32)
    h3 = jnp.maximum(h3 + b3[...][None, :], 0.0)
    wov = wo[...]
    z = jnp.dot(gm[...], wov[:NUM_FACTORS], preferred_element_type=jnp.float32)
    z += jnp.dot(h3, wov[NUM_FACTORS:], preferred_element_type=jnp.float32)
    out[...] = jax.nn.sigmoid(z)


def _tc_mlp(gm, um, vm, w1, b1, w2, b2, w3, b3, wo):
    grid = (BATCH // _R,)
    row_spec = pl.BlockSpec((_R, NUM_FACTORS), lambda i: (i, 0))
    full = lambda s: pl.BlockSpec(s, lambda i: (0,) * len(s))
    return pl.pallas_call(
        _tc_mlp_body,
        grid=grid,
        in_specs=[
            row_spec, row_spec, row_spec,
            full(w1.shape), full(b1.shape),
            full(w2.shape), full(b2.shape),
            full(w3.shape), full(b3.shape),
            full(wo.shape),
        ],
        out_specs=pl.BlockSpec((_R, 1), lambda i: (i, 0)),
        out_shape=jax.ShapeDtypeStruct((BATCH, 1), jnp.float32),
    )(gm, um, vm, w1, b1, w2, b2, w3, b3, wo)


def kernel(user_ids, item_ids, P, Q, U, V, W1, b1, W2, b2, W3, b3, Wo):
    gmf, p_mlp, q_mlp = _sc_gather(user_ids, item_ids, P, Q, U, V)
    return _tc_mlp(gmf, p_mlp, q_mlp, W1, b1, W2, b2, W3, b3, Wo)


# speedup vs baseline: 1.1002x; 1.1002x over previous
import functools

import jax
import jax.numpy as jnp
from jax import lax
from jax.experimental import pallas as pl
from jax.experimental.pallas import tpu as pltpu
from jax.experimental.pallas import tpu_sc as plsc

NUM_FACTORS = 128
BATCH = 16384

_SC_INFO = plsc.get_sparse_core_info()
_NC = _SC_INFO.num_cores
_NS = _SC_INFO.num_subcores
_NW = _NC * _NS
_B_PER_W = BATCH // _NW
_CHUNK = 64
_NCHUNKS = _B_PER_W // _CHUNK


def _sc_gather_body(uid, iid, p_hbm, q_hbm, u_hbm, v_hbm,
                    op, oq, ou, ov,
                    idx_u0, idx_i0, bp0, bq0, bu0, bv0,
                    idx_u1, idx_i1, bp1, bq1, bu1, bv1,
                    gsem0, gsem1, ssem0, ssem1):
    idx_u = (idx_u0, idx_u1)
    idx_i = (idx_i0, idx_i1)
    bp = (bp0, bp1)
    bq = (bq0, bq1)
    bu = (bu0, bu1)
    bv = (bv0, bv1)
    gsem = (gsem0, gsem1)
    ssem = (ssem0, ssem1)

    wid = lax.axis_index("s") * _NC + lax.axis_index("c")
    base = wid * _B_PER_W

    def issue_gathers(c, d):
        row0 = base + c * _CHUNK
        pltpu.sync_copy(uid.at[pl.ds(row0, _CHUNK)], idx_u[d])
        pltpu.sync_copy(iid.at[pl.ds(row0, _CHUNK)], idx_i[d])
        return (
            pltpu.async_copy(p_hbm.at[idx_u[d]], bp[d], gsem[d]),
            pltpu.async_copy(q_hbm.at[idx_i[d]], bq[d], gsem[d]),
            pltpu.async_copy(u_hbm.at[idx_u[d]], bu[d], gsem[d]),
            pltpu.async_copy(v_hbm.at[idx_i[d]], bv[d], gsem[d]),
        )

    def issue_scatters(c, d):
        row0 = base + c * _CHUNK
        return (
            pltpu.async_copy(bp[d], op.at[pl.ds(row0, _CHUNK)], ssem[d]),
            pltpu.async_copy(bq[d], oq.at[pl.ds(row0, _CHUNK)], ssem[d]),
            pltpu.async_copy(bu[d], ou.at[pl.ds(row0, _CHUNK)], ssem[d]),
            pltpu.async_copy(bv[d], ov.at[pl.ds(row0, _CHUNK)], ssem[d]),
        )

    gcopies = [None, None]
    scopies = [None, None]
    gcopies[0] = issue_gathers(0, 0)
    gcopies[1] = issue_gathers(1, 1)
    for c in range(_NCHUNKS):
        d = c % 2
        for cp in gcopies[d]:
            cp.wait()
        scopies[d] = issue_scatters(c, d)
        if c + 2 < _NCHUNKS:
            for cp in scopies[d]:
                cp.wait()
            gcopies[d] = issue_gathers(c + 2, d)
    for d in range(2):
        for cp in scopies[d]:
            cp.wait()


_ROW_SHAPE = jax.ShapeDtypeStruct((BATCH, NUM_FACTORS), jnp.float32)


def _row_bufs():
    return [
        pltpu.VMEM((_CHUNK,), jnp.int32),
        pltpu.VMEM((_CHUNK,), jnp.int32),
        pltpu.VMEM((_CHUNK, NUM_FACTORS), jnp.float32),
        pltpu.VMEM((_CHUNK, NUM_FACTORS), jnp.float32),
        pltpu.VMEM((_CHUNK, NUM_FACTORS), jnp.float32),
        pltpu.VMEM((_CHUNK, NUM_FACTORS), jnp.float32),
    ]


_sc_gather = functools.partial(
    pl.kernel,
    mesh=plsc.VectorSubcoreMesh(core_axis_name="c", subcore_axis_name="s"),
    out_type=(_ROW_SHAPE, _ROW_SHAPE, _ROW_SHAPE, _ROW_SHAPE),
    scratch_types=_row_bufs() + _row_bufs() + [
        pltpu.SemaphoreType.DMA,
        pltpu.SemaphoreType.DMA,
        pltpu.SemaphoreType.DMA,
        pltpu.SemaphoreType.DMA,
    ],
)(_sc_gather_body)


_R = 2048


def _tc_mlp_body(pm, qm, um, vm, w1, b1, w2, b2, w3, b3, wo, out):
    bf = jnp.bfloat16
    xu = um[...].astype(bf)
    xv = vm[...].astype(bf)
    w1v = w1[...].astype(bf)
    h1 = jnp.dot(xu, w1v[:NUM_FACTORS], preferred_element_type=jnp.float32)
    h1 += jnp.dot(xv, w1v[NUM_FACTORS:], preferred_element_type=jnp.float32)
    h1 = jnp.maximum(h1 + b1[...][None, :], 0.0).astype(bf)
    h2 = jnp.dot(h1, w2[...].astype(bf), preferred_element_type=jnp.float32)
    h2 = jnp.maximum(h2 + b2[...][None, :], 0.0).astype(bf)
    h3 = jnp.dot(h2, w3[...].astype(bf), preferred_element_type=jnp.float32)
    h3 = jnp.maximum(h3 + b3[...][None, :], 0.0)
    gmf = pm[...] * qm[...]
    wov = wo[...]
    z = jnp.dot(gmf, wov[:NUM_FACTORS], preferred_element_type=jnp.float32)
    z += jnp.dot(h3, wov[NUM_FACTORS:], preferred_element_type=jnp.float32)
    out[...] = jax.nn.sigmoid(z)


def _tc_mlp(pm, qm, um, vm, w1, b1, w2, b2, w3, b3, wo):
    grid = (BATCH // _R,)
    row_spec = pl.BlockSpec((_R, NUM_FACTORS), lambda i: (i, 0))
    full = lambda s: pl.BlockSpec(s, lambda i: (0,) * len(s))
    return pl.pallas_call(
        _tc_mlp_body,
        grid=grid,
        in_specs=[
            row_spec, row_spec, row_spec, row_spec,
            full(w1.shape), full(b1.shape),
            full(w2.shape), full(b2.shape),
            full(w3.shape), full(b3.shape),
            full(wo.shape),
        ],
        out_specs=pl.BlockSpec((_R, 1), lambda i: (i, 0)),
        out_shape=jax.ShapeDtypeStruct((BATCH, 1), jnp.float32),
    )(pm, qm, um, vm, w1, b1, w2, b2, w3, b3, wo)


def kernel(user_ids, item_ids, P, Q, U, V, W1, b1, W2, b2, W3, b3, Wo):
    p_mf, q_mf, p_mlp, q_mlp = _sc_gather(user_ids, item_ids, P, Q, U, V)
    return _tc_mlp(p_mf, q_mf, p_mlp, q_mlp, W1, b1, W2, b2, W3, b3, Wo)


# speedup vs baseline: 1.1311x; 1.0280x over previous
import functools

import jax
import jax.numpy as jnp
from jax import lax
from jax.experimental import pallas as pl
from jax.experimental.pallas import tpu as pltpu
from jax.experimental.pallas import tpu_sc as plsc

NUM_FACTORS = 128
BATCH = 16384
NSPLIT = 2

_SC_INFO = plsc.get_sparse_core_info()
_NC = _SC_INFO.num_cores
_NS = _SC_INFO.num_subcores
_NW = _NC * _NS
_CHUNK = 64


def _make_sc_gather(batch):
    b_per_w = batch // _NW
    nchunks = b_per_w // _CHUNK

    def body(uid, iid, p_hbm, q_hbm, u_hbm, v_hbm,
             op, oq, ou, ov,
             idx_u0, idx_i0, bp0, bq0, bu0, bv0,
             idx_u1, idx_i1, bp1, bq1, bu1, bv1,
             gsem0, gsem1, ssem0, ssem1):
        idx_u = (idx_u0, idx_u1)
        idx_i = (idx_i0, idx_i1)
        bp = (bp0, bp1)
        bq = (bq0, bq1)
        bu = (bu0, bu1)
        bv = (bv0, bv1)
        gsem = (gsem0, gsem1)
        ssem = (ssem0, ssem1)

        wid = lax.axis_index("s") * _NC + lax.axis_index("c")
        base = wid * b_per_w

        def issue_gathers(c, d):
            row0 = base + c * _CHUNK
            pltpu.sync_copy(uid.at[pl.ds(row0, _CHUNK)], idx_u[d])
            pltpu.sync_copy(iid.at[pl.ds(row0, _CHUNK)], idx_i[d])
            return (
                pltpu.async_copy(p_hbm.at[idx_u[d]], bp[d], gsem[d]),
                pltpu.async_copy(q_hbm.at[idx_i[d]], bq[d], gsem[d]),
                pltpu.async_copy(u_hbm.at[idx_u[d]], bu[d], gsem[d]),
                pltpu.async_copy(v_hbm.at[idx_i[d]], bv[d], gsem[d]),
            )

        def issue_scatters(c, d):
            row0 = base + c * _CHUNK
            return (
                pltpu.async_copy(bp[d], op.at[pl.ds(row0, _CHUNK)], ssem[d]),
                pltpu.async_copy(bq[d], oq.at[pl.ds(row0, _CHUNK)], ssem[d]),
                pltpu.async_copy(bu[d], ou.at[pl.ds(row0, _CHUNK)], ssem[d]),
                pltpu.async_copy(bv[d], ov.at[pl.ds(row0, _CHUNK)], ssem[d]),
            )

        gcopies = [None, None]
        scopies = [None, None]
        gcopies[0] = issue_gathers(0, 0)
        if nchunks > 1:
            gcopies[1] = issue_gathers(1, 1)
        for c in range(nchunks):
            d = c % 2
            for cp in gcopies[d]:
                cp.wait()
            scopies[d] = issue_scatters(c, d)
            if c + 2 < nchunks:
                for cp in scopies[d]:
                    cp.wait()
                gcopies[d] = issue_gathers(c + 2, d)
        for d in range(min(2, nchunks)):
            for cp in scopies[d]:
                cp.wait()

    row_shape = jax.ShapeDtypeStruct((batch, NUM_FACTORS), jnp.float32)

    def row_bufs():
        return [
            pltpu.VMEM((_CHUNK,), jnp.int32),
            pltpu.VMEM((_CHUNK,), jnp.int32),
            pltpu.VMEM((_CHUNK, NUM_FACTORS), jnp.float32),
            pltpu.VMEM((_CHUNK, NUM_FACTORS), jnp.float32),
            pltpu.VMEM((_CHUNK, NUM_FACTORS), jnp.float32),
            pltpu.VMEM((_CHUNK, NUM_FACTORS), jnp.float32),
        ]

    return functools.partial(
        pl.kernel,
        mesh=plsc.VectorSubcoreMesh(core_axis_name="c", subcore_axis_name="s"),
        out_type=(row_shape, row_shape, row_shape, row_shape),
        scratch_types=row_bufs() + row_bufs() + [
            pltpu.SemaphoreType.DMA,
            pltpu.SemaphoreType.DMA,
            pltpu.SemaphoreType.DMA,
            pltpu.SemaphoreType.DMA,
        ],
    )(body)


_R = 2048


def _tc_mlp_body(pm, qm, um, vm, w1, b1, w2, b2, w3, b3, wo, out):
    bf = jnp.bfloat16
    xu = um[...].astype(bf)
    xv = vm[...].astype(bf)
    w1v = w1[...].astype(bf)
    h1 = jnp.dot(xu, w1v[:NUM_FACTORS], preferred_element_type=jnp.float32)
    h1 += jnp.dot(xv, w1v[NUM_FACTORS:], preferred_element_type=jnp.float32)
    h1 = jnp.maximum(h1 + b1[...][None, :], 0.0).astype(bf)
    h2 = jnp.dot(h1, w2[...].astype(bf), preferred_element_type=jnp.float32)
    h2 = jnp.maximum(h2 + b2[...][None, :], 0.0).astype(bf)
    h3 = jnp.dot(h2, w3[...].astype(bf), preferred_element_type=jnp.float32)
    h3 = jnp.maximum(h3 + b3[...][None, :], 0.0)
    gmf = pm[...] * qm[...]
    wov = wo[...]
    z = jnp.dot(gmf, wov[:NUM_FACTORS], preferred_element_type=jnp.float32)
    z += jnp.dot(h3, wov[NUM_FACTORS:], preferred_element_type=jnp.float32)
    out[...] = jax.nn.sigmoid(z)


def _tc_mlp(pm, qm, um, vm, w1, b1, w2, b2, w3, b3, wo):
    batch = pm.shape[0]
    grid = (batch // _R,)
    row_spec = pl.BlockSpec((_R, NUM_FACTORS), lambda i: (i, 0))
    full = lambda s: pl.BlockSpec(s, lambda i: (0,) * len(s))
    return pl.pallas_call(
        _tc_mlp_body,
        grid=grid,
        in_specs=[
            row_spec, row_spec, row_spec, row_spec,
            full(w1.shape), full(b1.shape),
            full(w2.shape), full(b2.shape),
            full(w3.shape), full(b3.shape),
            full(wo.shape),
        ],
        out_specs=pl.BlockSpec((_R, 1), lambda i: (i, 0)),
        out_shape=jax.ShapeDtypeStruct((batch, 1), jnp.float32),
    )(pm, qm, um, vm, w1, b1, w2, b2, w3, b3, wo)


def kernel(user_ids, item_ids, P, Q, U, V, W1, b1, W2, b2, W3, b3, Wo):
    half = BATCH // NSPLIT
    sc = _make_sc_gather(half)
    gathered = []
    for s in range(NSPLIT):
        sl = slice(s * half, (s + 1) * half)
        gathered.append(sc(user_ids[sl], item_ids[sl], P, Q, U, V))
    outs = [
        _tc_mlp(pm, qm, um, vm, W1, b1, W2, b2, W3, b3, Wo)
        for (pm, qm, um, vm) in gathered
    ]
    return jnp.concatenate(outs, axis=0)
